# baseline (speedup 1.0000x reference)
import functools

import jax
import jax.numpy as jnp
from jax import lax
from jax.experimental import pallas as pl
from jax.experimental.pallas import tpu as pltpu
from jax.experimental.pallas import tpu_sc as plsc

_NW = 32
_C = 32
_NBUF = 3


def _embed_flat(idx3, table):
    n_chunks = idx3.shape[1]
    b_per_w = n_chunks * _C
    B = _NW * b_per_w
    D = table.shape[1]
    mesh = plsc.VectorSubcoreMesh(core_axis_name="c", subcore_axis_name="s")

    @functools.partial(
        pl.kernel,
        mesh=mesh,
        out_type=jax.ShapeDtypeStruct((B, D), jnp.float32),
        scratch_types=(
            [pltpu.VMEM((n_chunks, _C), jnp.int32),
             pltpu.VMEM((_NBUF, _C, D), jnp.float32)]
            + [pltpu.SemaphoreType.DMA] * (2 * _NBUF)
        ),
    )
    def k(idx_hbm, table_hbm, out_hbm, idx_v, rows_v, *sems):
        gsems = sems[:_NBUF]
        ssems = sems[_NBUF:]

        cid = lax.axis_index("c")
        sid = lax.axis_index("s")
        wid = sid * 2 + cid
        base = wid * b_per_w

        pltpu.sync_copy(idx_hbm.at[wid], idx_v)

        def gather(j, buf):
            return pltpu.async_copy(
                table_hbm.at[idx_v.at[j]], rows_v.at[buf], gsems[buf])

        def store(j, buf):
            return pltpu.async_copy(
                rows_v.at[buf], out_hbm.at[pl.ds(base + j * _C, _C)],
                ssems[buf])

        gd = [None] * n_chunks
        sd = [None] * n_chunks
        waited = [False] * n_chunks
        for b in range(min(_NBUF - 1, n_chunks)):
            gd[b] = gather(b, b)
        for j in range(n_chunks):
            buf = j % _NBUF
            gd[j].wait()
            sd[j] = store(j, buf)
            nj = j + _NBUF - 1
            if nj < n_chunks:
                prev = nj - _NBUF
                if prev >= 0:
                    sd[prev].wait()
                    waited[prev] = True
                gd[nj] = gather(nj, nj % _NBUF)
        for j in range(n_chunks):
            if not waited[j]:
                sd[j].wait()

    return k(idx3, table)


def kernel(token_ids, table):
    S0, S1 = token_ids.shape
    B = S0 * S1
    n_chunks = B // (_NW * _C)
    idx3 = token_ids.reshape(_NW, n_chunks, _C).astype(jnp.int32)
    out = _embed_flat(idx3, table)
    return out.reshape(S0, S1, table.shape[1])


# speedup vs baseline: 1.0088x; 1.0088x over previous
import functools

import jax
import jax.numpy as jnp
from jax import lax
from jax.experimental import pallas as pl
from jax.experimental.pallas import tpu as pltpu
from jax.experimental.pallas import tpu_sc as plsc

_NW = 32
_C = 32
_NBUF = 3


def _embed_flat(idx3, table):
    n_chunks = idx3.shape[1]
    b_per_w = n_chunks * _C
    B = _NW * b_per_w
    D = table.shape[1]
    mesh = plsc.VectorSubcoreMesh(core_axis_name="c", subcore_axis_name="s")

    @functools.partial(
        pl.kernel,
        mesh=mesh,
        out_type=jax.ShapeDtypeStruct((B, D), jnp.float32),
        scratch_types=[
            pltpu.VMEM((n_chunks, _C), jnp.int32),
            pltpu.VMEM((_NBUF, _C, D), jnp.float32),
        ] + [pltpu.SemaphoreType.DMA] * (2 * _NBUF),
    )
    def k(idx_hbm, table_hbm, out_hbm, idx_v, rows_v, *sems):
        cid = lax.axis_index("c")
        sid = lax.axis_index("s")
        wid = sid * 2 + cid
        base = wid * b_per_w

        pltpu.sync_copy(idx_hbm.at[wid], idx_v)

        gsems = sems[:_NBUF]
        ssems = sems[_NBUF:]

        def gather(j, buf):
            return pltpu.async_copy(
                table_hbm.at[idx_v.at[j]], rows_v.at[buf], gsems[buf])

        def store(j, buf):
            return pltpu.async_copy(
                rows_v.at[buf], out_hbm.at[pl.ds(base + j * _C, _C)],
                ssems[buf])

        gd = [None] * n_chunks
        sd = [None] * n_chunks
        waited = [False] * n_chunks
        for b in range(min(_NBUF - 1, n_chunks)):
            gd[b] = gather(b, b)
        for j in range(n_chunks):
            buf = j % _NBUF
            nj = j + _NBUF - 1
            if nj < n_chunks:
                prev = nj - _NBUF
                if prev >= 0:
                    sd[prev].wait()
                    waited[prev] = True
                gd[nj] = gather(nj, nj % _NBUF)
            gd[j].wait()
            sd[j] = store(j, buf)
        for j in range(n_chunks):
            if not waited[j]:
                sd[j].wait()

    return k(idx3, table)


def kernel(token_ids, table):
    S0, S1 = token_ids.shape
    B = S0 * S1
    n_chunks = B // (_NW * _C)
    idx3 = token_ids.reshape(_NW, n_chunks, _C).astype(jnp.int32)
    out = _embed_flat(idx3, table)
    return out.reshape(S0, S1, table.shape[1])


# speedup vs baseline: 1.0132x; 1.0044x over previous
import functools

import jax
import jax.numpy as jnp
from jax import lax
from jax.experimental import pallas as pl
from jax.experimental.pallas import tpu as pltpu
from jax.experimental.pallas import tpu_sc as plsc

_NW = 32
_C = 16
_NBUF = 7


def _embed_flat(idx3, table):
    n_chunks = idx3.shape[1]
    b_per_w = n_chunks * _C
    B = _NW * b_per_w
    D = table.shape[1]
    mesh = plsc.VectorSubcoreMesh(core_axis_name="c", subcore_axis_name="s")

    @functools.partial(
        pl.kernel,
        mesh=mesh,
        out_type=jax.ShapeDtypeStruct((B, D), jnp.float32),
        scratch_types=[
            pltpu.VMEM((n_chunks, _C), jnp.int32),
            pltpu.VMEM((_NBUF, _C, D), jnp.float32),
        ] + [pltpu.SemaphoreType.DMA] * (2 * _NBUF),
    )
    def k(idx_hbm, table_hbm, out_hbm, idx_v, rows_v, *sems):
        cid = lax.axis_index("c")
        sid = lax.axis_index("s")
        wid = sid * 2 + cid
        base = wid * b_per_w

        pltpu.sync_copy(idx_hbm.at[wid], idx_v)

        gsems = sems[:_NBUF]
        ssems = sems[_NBUF:]

        def gather(j, buf):
            return pltpu.async_copy(
                table_hbm.at[idx_v.at[j]], rows_v.at[buf], gsems[buf])

        def store(j, buf):
            return pltpu.async_copy(
                rows_v.at[buf], out_hbm.at[pl.ds(base + j * _C, _C)],
                ssems[buf])

        gd = [None] * n_chunks
        sd = [None] * n_chunks
        waited = [False] * n_chunks
        for b in range(min(_NBUF - 1, n_chunks)):
            gd[b] = gather(b, b)
        for j in range(n_chunks):
            buf = j % _NBUF
            nj = j + _NBUF - 1
            if nj < n_chunks:
                prev = nj - _NBUF
                if prev >= 0:
                    sd[prev].wait()
                    waited[prev] = True
                gd[nj] = gather(nj, nj % _NBUF)
            gd[j].wait()
            sd[j] = store(j, buf)
        for j in range(n_chunks):
            if not waited[j]:
                sd[j].wait()

    return k(idx3, table)


def kernel(token_ids, table):
    S0, S1 = token_ids.shape
    B = S0 * S1
    n_chunks = B // (_NW * _C)
    idx3 = token_ids.reshape(_NW, n_chunks, _C).astype(jnp.int32)
    out = _embed_flat(idx3, table)
    return out.reshape(S0, S1, table.shape[1])
